# baseline (speedup 1.0000x reference)
import functools

import jax
import jax.numpy as jnp
from jax import lax
from jax.experimental import pallas as pl
from jax.experimental.pallas import tpu as pltpu
from jax.experimental.pallas import tpu_sc as plsc

N = 10000
E = 320000
D_IN = 128
D_H = 128
D_OUT = 64

NC = 2
NS = 16
NW = NC * NS

N_PAD = 10112
RPT = N_PAD // NS
CHUNK = 96
K = 108
KH = K // 2
EPW = K * CHUNK
E_PAD = EPW * NW

_MESH = plsc.VectorSubcoreMesh(core_axis_name="c", subcore_axis_name="s")
_SC_PARAMS = pltpu.CompilerParams(use_tc_tiling_on_sc=False)


def _sc_degree(src_hbm, dst_hbm, ones_hbm, zrow_hbm, out_hbm,
               src_v, dst_v, ones_v, acc_o, acc_i, sem):
    cid = lax.axis_index("c")
    sid = lax.axis_index("s")
    wid = cid * NS + sid

    pltpu.sync_copy(ones_hbm, ones_v)
    pltpu.sync_copy(zrow_hbm, acc_o.at[pl.ds(sid * RPT, RPT)])
    pltpu.sync_copy(zrow_hbm, acc_i.at[pl.ds(sid * RPT, RPT)])
    pltpu.sync_copy(src_hbm.at[wid], src_v)
    pltpu.sync_copy(dst_hbm.at[wid], dst_v)
    plsc.subcore_barrier()

    def body(j, c):
        pltpu.sync_copy(ones_v, acc_o.at[src_v.at[j]], add=True)
        pltpu.sync_copy(ones_v, acc_i.at[dst_v.at[j]], add=True)
        return c

    lax.fori_loop(0, K, body, 0)
    plsc.subcore_barrier()

    pltpu.sync_copy(acc_o.at[pl.ds(sid * RPT, RPT)],
                    out_hbm.at[pl.ds(cid * 2 * N_PAD + sid * RPT, RPT)])
    pltpu.sync_copy(acc_i.at[pl.ds(sid * RPT, RPT)],
                    out_hbm.at[pl.ds(cid * 2 * N_PAD + N_PAD + sid * RPT, RPT)])


_deg_call = functools.partial(
    pl.kernel,
    _sc_degree,
    out_type=jax.ShapeDtypeStruct((NC * 2 * N_PAD,), jnp.float32),
    mesh=_MESH,
    scratch_types=[
        pltpu.VMEM((K, CHUNK), jnp.int32),
        pltpu.VMEM((K, CHUNK), jnp.int32),
        pltpu.VMEM((CHUNK,), jnp.float32),
        pltpu.VMEM_SHARED((N_PAD,), jnp.float32),
        pltpu.VMEM_SHARED((N_PAD,), jnp.float32),
        pltpu.SemaphoreType.DMA,
    ],
    compiler_params=_SC_PARAMS,
)()


def _make_sc_scatter(d):
    def body(src_hbm, dst_hbm, y_hbm, zrows_hbm, out_hbm,
             src_v, dst_v, rows_a, rows_b, acc, sem_z, sem_i, sem_a, sem_b):
        cid = lax.axis_index("c")
        sid = lax.axis_index("s")
        wid = cid * NS + sid

        pltpu.async_copy(zrows_hbm.at[sid], acc.at[pl.ds(sid * RPT, RPT)],
                         sem_z)
        pltpu.async_copy(src_hbm.at[wid], src_v, sem_i)
        pltpu.async_copy(dst_hbm.at[wid], dst_v, sem_i)
        pltpu.make_async_copy(src_hbm.at[wid], src_v, sem_i).wait()
        pltpu.make_async_copy(dst_hbm.at[wid], dst_v, sem_i).wait()
        pltpu.make_async_copy(zrows_hbm.at[sid],
                              acc.at[pl.ds(sid * RPT, RPT)], sem_z).wait()
        plsc.subcore_barrier()

        pltpu.async_copy(y_hbm.at[src_v.at[0]], rows_a, sem_a)

        def body2(t, c):
            a = 2 * t
            b = a + 1
            pltpu.make_async_copy(y_hbm.at[src_v.at[a]], rows_a, sem_a).wait()
            pltpu.async_copy(y_hbm.at[src_v.at[b]], rows_b, sem_b)
            pltpu.sync_copy(rows_a, acc.at[dst_v.at[a]], add=True)
            pltpu.make_async_copy(y_hbm.at[src_v.at[b]], rows_b, sem_b).wait()

            @pl.when(t < KH - 1)
            def _():
                pltpu.async_copy(y_hbm.at[src_v.at[a + 2]], rows_a, sem_a)

            pltpu.sync_copy(rows_b, acc.at[dst_v.at[b]], add=True)
            return c

        lax.fori_loop(0, KH, body2, 0)
        plsc.subcore_barrier()

        pltpu.sync_copy(acc.at[pl.ds(sid * RPT, RPT)],
                        out_hbm.at[pl.ds(cid * N_PAD + sid * RPT, RPT)])

    return functools.partial(
        pl.kernel,
        body,
        out_type=jax.ShapeDtypeStruct((NC * N_PAD, d), jnp.float32),
        mesh=_MESH,
        scratch_types=[
            pltpu.VMEM((K, CHUNK), jnp.int32),
            pltpu.VMEM((K, CHUNK), jnp.int32),
            pltpu.VMEM((CHUNK, d), jnp.float32),
            pltpu.VMEM((CHUNK, d), jnp.float32),
            pltpu.VMEM_SHARED((N_PAD, d), jnp.float32),
            pltpu.SemaphoreType.DMA,
            pltpu.SemaphoreType.DMA,
            pltpu.SemaphoreType.DMA,
            pltpu.SemaphoreType.DMA,
        ],
        compiler_params=_SC_PARAMS,
    )()


_sc_scatter_h = _make_sc_scatter(D_H)
_sc_scatter_o = _make_sc_scatter(D_OUT)

B_R = 1264
_GRID = (N_PAD // B_R,)


def _tc0_body(x_ref, doa_ref, dob_ref, w_ref, o_ref):
    s = lax.rsqrt(jnp.maximum(doa_ref[...] + dob_ref[...], 1.0))
    o_ref[...] = jnp.dot(x_ref[...] * s, w_ref[...],
                         preferred_element_type=jnp.float32)


def _tc_mid_body(p0_ref, p1_ref, dia_ref, dib_ref, b_ref, doa_ref, dob_ref,
                 w_ref, o_ref):
    si = lax.rsqrt(jnp.maximum(dia_ref[...] + dib_ref[...], 1.0))
    h = (p0_ref[...] + p1_ref[...]) * si + b_ref[...]
    h = jnp.maximum(h, 0.0)
    so = lax.rsqrt(jnp.maximum(doa_ref[...] + dob_ref[...], 1.0))
    o_ref[...] = jnp.dot(h * so, w_ref[...],
                         preferred_element_type=jnp.float32)


def _tc_last_body(p0_ref, p1_ref, dia_ref, dib_ref, b_ref, o_ref):
    si = lax.rsqrt(jnp.maximum(dia_ref[...] + dib_ref[...], 1.0))
    o_ref[...] = (p0_ref[...] + p1_ref[...]) * si + b_ref[...]


def _row_spec(d):
    return pl.BlockSpec((B_R, d), lambda i: (i, 0))


def _full_spec(r, c):
    return pl.BlockSpec((r, c), lambda i: (0, 0))


_VEC = pl.BlockSpec((B_R, 1), lambda i: (i, 0))


def _tc0(x, doa, dob, w):
    d_in, d_out = w.shape
    return pl.pallas_call(
        _tc0_body,
        grid=_GRID,
        in_specs=[_row_spec(d_in), _VEC, _VEC, _full_spec(d_in, d_out)],
        out_specs=_row_spec(d_out),
        out_shape=jax.ShapeDtypeStruct((N_PAD, d_out), jnp.float32),
    )(x, doa, dob, w)


def _tc_mid(p0, p1, dia, dib, b, doa, dob, w):
    d_in, d_out = w.shape
    return pl.pallas_call(
        _tc_mid_body,
        grid=_GRID,
        in_specs=[_row_spec(d_in), _row_spec(d_in), _VEC, _VEC,
                  _full_spec(1, d_in), _VEC, _VEC, _full_spec(d_in, d_out)],
        out_specs=_row_spec(d_out),
        out_shape=jax.ShapeDtypeStruct((N_PAD, d_out), jnp.float32),
    )(p0, p1, dia, dib, b, doa, dob, w)


def _tc_last(p0, p1, dia, dib, b):
    d = p0.shape[1]
    return pl.pallas_call(
        _tc_last_body,
        grid=_GRID,
        in_specs=[_row_spec(d), _row_spec(d), _VEC, _VEC, _full_spec(1, d)],
        out_specs=_row_spec(d),
        out_shape=jax.ShapeDtypeStruct((N_PAD, d), jnp.float32),
    )(p0, p1, dia, dib, b)


def kernel(features, edge_index, W0, b0, W1, b1, W2, b2):
    x = jnp.pad(features, ((0, N_PAD - N), (0, 0)))
    pad_e = E_PAD - E
    src_p = jnp.concatenate(
        [edge_index[0], jnp.full((pad_e,), N, jnp.int32)]
    ).reshape(NW, K, CHUNK)
    dst_p = jnp.concatenate(
        [edge_index[1], jnp.full((pad_e,), N_PAD - 1, jnp.int32)]
    ).reshape(NW, K, CHUNK)
    ones_c = jnp.ones((CHUNK,), jnp.float32)
    zrow = jnp.zeros((RPT,), jnp.float32)
    zrows_h = jnp.zeros((NS, RPT, D_H), jnp.float32)
    zrows_o = jnp.zeros((NS, RPT, D_OUT), jnp.float32)

    deg = _deg_call(src_p, dst_p, ones_c, zrow).reshape(NC, 2, N_PAD)
    doa = deg[0, 0].reshape(N_PAD, 1)
    dob = deg[1, 0].reshape(N_PAD, 1)
    dia = deg[0, 1].reshape(N_PAD, 1)
    dib = deg[1, 1].reshape(N_PAD, 1)

    y = _tc0(x, doa, dob, W0)
    p = _sc_scatter_h(src_p, dst_p, y, zrows_h)
    y = _tc_mid(p[:N_PAD], p[N_PAD:], dia, dib, b0.reshape(1, D_H), doa, dob, W1)
    p = _sc_scatter_h(src_p, dst_p, y, zrows_h)
    y = _tc_mid(p[:N_PAD], p[N_PAD:], dia, dib, b1.reshape(1, D_H), doa, dob, W2)
    p = _sc_scatter_o(src_p, dst_p, y, zrows_o)
    out = _tc_last(p[:N_PAD], p[N_PAD:], dia, dib, b2.reshape(1, D_OUT))
    return out[:N]


# speedup vs baseline: 3.2775x; 3.2775x over previous
import functools

import jax
import jax.numpy as jnp
from jax import lax
from jax.experimental import pallas as pl
from jax.experimental.pallas import tpu as pltpu
from jax.experimental.pallas import tpu_sc as plsc

N = 10000
E = 320000
D_IN = 128
D_H = 128
D_OUT = 64

NC = 2
NS = 16
NW = NC * NS

N_PAD = 10112
RPT = N_PAD // NS
CHUNK = 80
K = 125
KH = (K - 1) // 2
EPW = K * CHUNK


_MESH = plsc.VectorSubcoreMesh(core_axis_name="c", subcore_axis_name="s")
_SC_PARAMS = pltpu.CompilerParams(use_tc_tiling_on_sc=False)


def _sc_degree(src_hbm, dst_hbm, ones_hbm, zrow_hbm, out_hbm,
               src_v, dst_v, ones_v, acc_o, acc_i, sem):
    cid = lax.axis_index("c")
    sid = lax.axis_index("s")
    wid = cid * NS + sid

    pltpu.sync_copy(ones_hbm, ones_v)
    pltpu.sync_copy(zrow_hbm, acc_o.at[pl.ds(sid * RPT, RPT)])
    pltpu.sync_copy(zrow_hbm, acc_i.at[pl.ds(sid * RPT, RPT)])
    pltpu.sync_copy(src_hbm.at[wid], src_v)
    pltpu.sync_copy(dst_hbm.at[wid], dst_v)
    plsc.subcore_barrier()

    def body(j, c):
        pltpu.sync_copy(ones_v, acc_o.at[src_v.at[j]], add=True)
        pltpu.sync_copy(ones_v, acc_i.at[dst_v.at[j]], add=True)
        return c

    lax.fori_loop(0, K, body, 0)
    plsc.subcore_barrier()

    pltpu.sync_copy(acc_o.at[pl.ds(sid * RPT, RPT)],
                    out_hbm.at[pl.ds(cid * 2 * N_PAD + sid * RPT, RPT)])
    pltpu.sync_copy(acc_i.at[pl.ds(sid * RPT, RPT)],
                    out_hbm.at[pl.ds(cid * 2 * N_PAD + N_PAD + sid * RPT, RPT)])


_deg_call = functools.partial(
    pl.kernel,
    _sc_degree,
    out_type=jax.ShapeDtypeStruct((NC * 2 * N_PAD,), jnp.float32),
    mesh=_MESH,
    scratch_types=[
        pltpu.VMEM((K, CHUNK), jnp.int32),
        pltpu.VMEM((K, CHUNK), jnp.int32),
        pltpu.VMEM((CHUNK,), jnp.float32),
        pltpu.VMEM_SHARED((N_PAD,), jnp.float32),
        pltpu.VMEM_SHARED((N_PAD,), jnp.float32),
        pltpu.SemaphoreType.DMA,
    ],
    compiler_params=_SC_PARAMS,
)()


def _make_sc_scatter(d):
    def body(src_hbm, dst_hbm, y_hbm, zrows_hbm, out_hbm,
             src_v, dst_v, rows_a, rows_b, acc, sem_z, sem_i, sem_a, sem_b):
        cid = lax.axis_index("c")
        sid = lax.axis_index("s")
        wid = cid * NS + sid

        pltpu.async_copy(zrows_hbm.at[sid], acc.at[pl.ds(sid * RPT, RPT)],
                         sem_z)
        pltpu.async_copy(src_hbm.at[wid], src_v, sem_i)
        pltpu.async_copy(dst_hbm.at[wid], dst_v, sem_i)
        pltpu.make_async_copy(src_hbm.at[wid], src_v, sem_i).wait()
        pltpu.make_async_copy(dst_hbm.at[wid], dst_v, sem_i).wait()
        pltpu.make_async_copy(zrows_hbm.at[sid],
                              acc.at[pl.ds(sid * RPT, RPT)], sem_z).wait()
        plsc.subcore_barrier()

        pltpu.async_copy(y_hbm.at[src_v.at[0]], rows_a, sem_a)

        def body2(t, c):
            a = 2 * t
            b = a + 1
            pltpu.make_async_copy(y_hbm.at[src_v.at[a]], rows_a, sem_a).wait()
            pltpu.async_copy(y_hbm.at[src_v.at[b]], rows_b, sem_b)
            pltpu.sync_copy(rows_a, acc.at[dst_v.at[a]], add=True)
            pltpu.make_async_copy(y_hbm.at[src_v.at[b]], rows_b, sem_b).wait()
            pltpu.async_copy(y_hbm.at[src_v.at[a + 2]], rows_a, sem_a)
            pltpu.sync_copy(rows_b, acc.at[dst_v.at[b]], add=True)
            return c

        lax.fori_loop(0, KH, body2, 0)
        pltpu.make_async_copy(y_hbm.at[src_v.at[K - 1]], rows_a, sem_a).wait()
        pltpu.sync_copy(rows_a, acc.at[dst_v.at[K - 1]], add=True)
        plsc.subcore_barrier()

        pltpu.sync_copy(acc.at[pl.ds(sid * RPT, RPT)],
                        out_hbm.at[pl.ds(cid * N_PAD + sid * RPT, RPT)])

    return functools.partial(
        pl.kernel,
        body,
        out_type=jax.ShapeDtypeStruct((NC * N_PAD, d), jnp.float32),
        mesh=_MESH,
        scratch_types=[
            pltpu.VMEM((K, CHUNK), jnp.int32),
            pltpu.VMEM((K, CHUNK), jnp.int32),
            pltpu.VMEM((CHUNK, d), jnp.float32),
            pltpu.VMEM((CHUNK, d), jnp.float32),
            pltpu.VMEM_SHARED((N_PAD, d), jnp.float32),
            pltpu.SemaphoreType.DMA,
            pltpu.SemaphoreType.DMA,
            pltpu.SemaphoreType.DMA,
            pltpu.SemaphoreType.DMA,
        ],
        compiler_params=_SC_PARAMS,
    )()


_sc_scatter_h = _make_sc_scatter(D_H)
_sc_scatter_o = _make_sc_scatter(D_OUT)

B_R = 1264
_GRID = (N_PAD // B_R,)


def _tc0_body(x_ref, doa_ref, dob_ref, w_ref, o_ref):
    s = lax.rsqrt(jnp.maximum(doa_ref[...] + dob_ref[...], 1.0))
    o_ref[...] = jnp.dot(x_ref[...] * s, w_ref[...],
                         preferred_element_type=jnp.float32)


def _tc_mid_body(p0_ref, p1_ref, dia_ref, dib_ref, b_ref, doa_ref, dob_ref,
                 w_ref, o_ref):
    si = lax.rsqrt(jnp.maximum(dia_ref[...] + dib_ref[...], 1.0))
    h = (p0_ref[...] + p1_ref[...]) * si + b_ref[...]
    h = jnp.maximum(h, 0.0)
    so = lax.rsqrt(jnp.maximum(doa_ref[...] + dob_ref[...], 1.0))
    o_ref[...] = jnp.dot(h * so, w_ref[...],
                         preferred_element_type=jnp.float32)


def _tc_last_body(p0_ref, p1_ref, dia_ref, dib_ref, b_ref, o_ref):
    si = lax.rsqrt(jnp.maximum(dia_ref[...] + dib_ref[...], 1.0))
    o_ref[...] = (p0_ref[...] + p1_ref[...]) * si + b_ref[...]


def _row_spec(d):
    return pl.BlockSpec((B_R, d), lambda i: (i, 0))


def _full_spec(r, c):
    return pl.BlockSpec((r, c), lambda i: (0, 0))


_VEC = pl.BlockSpec((B_R, 1), lambda i: (i, 0))


def _tc0(x, doa, dob, w):
    d_in, d_out = w.shape
    return pl.pallas_call(
        _tc0_body,
        grid=_GRID,
        in_specs=[_row_spec(d_in), _VEC, _VEC, _full_spec(d_in, d_out)],
        out_specs=_row_spec(d_out),
        out_shape=jax.ShapeDtypeStruct((N_PAD, d_out), jnp.float32),
    )(x, doa, dob, w)


def _tc_mid(p0, p1, dia, dib, b, doa, dob, w):
    d_in, d_out = w.shape
    return pl.pallas_call(
        _tc_mid_body,
        grid=_GRID,
        in_specs=[_row_spec(d_in), _row_spec(d_in), _VEC, _VEC,
                  _full_spec(1, d_in), _VEC, _VEC, _full_spec(d_in, d_out)],
        out_specs=_row_spec(d_out),
        out_shape=jax.ShapeDtypeStruct((N_PAD, d_out), jnp.float32),
    )(p0, p1, dia, dib, b, doa, dob, w)


def _tc_last(p0, p1, dia, dib, b):
    d = p0.shape[1]
    return pl.pallas_call(
        _tc_last_body,
        grid=_GRID,
        in_specs=[_row_spec(d), _row_spec(d), _VEC, _VEC, _full_spec(1, d)],
        out_specs=_row_spec(d),
        out_shape=jax.ShapeDtypeStruct((N_PAD, d), jnp.float32),
    )(p0, p1, dia, dib, b)


def kernel(features, edge_index, W0, b0, W1, b1, W2, b2):
    x = jnp.pad(features, ((0, N_PAD - N), (0, 0)))
    src_p = edge_index[0].reshape(NW, K, CHUNK)
    dst_p = edge_index[1].reshape(NW, K, CHUNK)
    ones_c = jnp.ones((CHUNK,), jnp.float32)
    zrow = jnp.zeros((RPT,), jnp.float32)
    zrows_h = jnp.zeros((NS, RPT, D_H), jnp.float32)
    zrows_o = jnp.zeros((NS, RPT, D_OUT), jnp.float32)

    deg = _deg_call(src_p, dst_p, ones_c, zrow).reshape(NC, 2, N_PAD)
    doa = deg[0, 0].reshape(N_PAD, 1)
    dob = deg[1, 0].reshape(N_PAD, 1)
    dia = deg[0, 1].reshape(N_PAD, 1)
    dib = deg[1, 1].reshape(N_PAD, 1)

    y = _tc0(x, doa, dob, W0)
    p = _sc_scatter_h(src_p, dst_p, y, zrows_h)
    y = _tc_mid(p[:N_PAD], p[N_PAD:], dia, dib, b0.reshape(1, D_H), doa, dob, W1)
    p = _sc_scatter_h(src_p, dst_p, y, zrows_h)
    y = _tc_mid(p[:N_PAD], p[N_PAD:], dia, dib, b1.reshape(1, D_H), doa, dob, W2)
    p = _sc_scatter_o(src_p, dst_p, y, zrows_o)
    out = _tc_last(p[:N_PAD], p[N_PAD:], dia, dib, b2.reshape(1, D_OUT))
    return out[:N]


# speedup vs baseline: 3.4364x; 1.0485x over previous
import functools

import jax
import jax.numpy as jnp
from jax import lax
from jax.experimental import pallas as pl
from jax.experimental.pallas import tpu as pltpu
from jax.experimental.pallas import tpu_sc as plsc

N = 10000
E = 320000
D_IN = 128
D_H = 128
D_OUT = 64

NC = 2
NS = 16
NW = NC * NS

RPT = N // NS
CHUNK = 80
K = 125
KH = (K - 1) // 2
EPW = K * CHUNK
DTILES = 10

_MESH = plsc.VectorSubcoreMesh(core_axis_name="c", subcore_axis_name="s")
_SC_PARAMS = pltpu.CompilerParams(use_tc_tiling_on_sc=False)


def _sc_degree(eidx_hbm, ones_hbm, zrow_hbm, out_hbm,
               idx_v, ones_v, acc_o, acc_i, sem):
    cid = lax.axis_index("c")
    sid = lax.axis_index("s")
    wid = cid * NS + sid

    pltpu.sync_copy(ones_hbm, ones_v)

    @pl.when(sid < DTILES)
    def _():
        pltpu.sync_copy(zrow_hbm, acc_o.at[pl.ds(sid * 1000, 1000)])
        pltpu.sync_copy(zrow_hbm, acc_i.at[pl.ds(sid * 1000, 1000)])

    pltpu.sync_copy(eidx_hbm.at[0, wid], idx_v.at[0])
    pltpu.sync_copy(eidx_hbm.at[1, wid], idx_v.at[1])
    plsc.subcore_barrier()

    def body(j, c):
        pltpu.sync_copy(ones_v, acc_o.at[idx_v.at[0, j]], add=True)
        pltpu.sync_copy(ones_v, acc_i.at[idx_v.at[1, j]], add=True)
        return c

    lax.fori_loop(0, K, body, 0)
    plsc.subcore_barrier()

    @pl.when(sid < DTILES)
    def _():
        pltpu.sync_copy(acc_o.at[pl.ds(sid * 1000, 1000)],
                        out_hbm.at[pl.ds(cid * 2 * N + sid * 1000, 1000)])
        pltpu.sync_copy(acc_i.at[pl.ds(sid * 1000, 1000)],
                        out_hbm.at[pl.ds(cid * 2 * N + N + sid * 1000, 1000)])


_deg_call = functools.partial(
    pl.kernel,
    _sc_degree,
    out_type=jax.ShapeDtypeStruct((NC * 2 * N,), jnp.float32),
    mesh=_MESH,
    scratch_types=[
        pltpu.VMEM((2, K, CHUNK), jnp.int32),
        pltpu.VMEM((CHUNK,), jnp.float32),
        pltpu.VMEM_SHARED((N,), jnp.float32),
        pltpu.VMEM_SHARED((N,), jnp.float32),
        pltpu.SemaphoreType.DMA,
    ],
    compiler_params=_SC_PARAMS,
)()


def _make_sc_scatter(d):
    def body(eidx_hbm, y_hbm, zrows_hbm, out_hbm,
             idx_v, rows_a, rows_b, acc, sem_z, sem_i, sem_a, sem_b):
        cid = lax.axis_index("c")
        sid = lax.axis_index("s")
        wid = cid * NS + sid

        pltpu.async_copy(zrows_hbm.at[sid], acc.at[pl.ds(sid * RPT, RPT)],
                         sem_z)
        pltpu.async_copy(eidx_hbm.at[0, wid], idx_v.at[0], sem_i)
        pltpu.async_copy(eidx_hbm.at[1, wid], idx_v.at[1], sem_i)
        pltpu.make_async_copy(eidx_hbm.at[0, wid], idx_v.at[0], sem_i).wait()
        pltpu.make_async_copy(eidx_hbm.at[1, wid], idx_v.at[1], sem_i).wait()
        pltpu.make_async_copy(zrows_hbm.at[sid],
                              acc.at[pl.ds(sid * RPT, RPT)], sem_z).wait()
        plsc.subcore_barrier()

        src_v = idx_v.at[0]
        dst_v = idx_v.at[1]
        pltpu.async_copy(y_hbm.at[src_v.at[0]], rows_a, sem_a)

        def body2(t, c):
            a = 2 * t
            b = a + 1
            pltpu.make_async_copy(y_hbm.at[src_v.at[a]], rows_a, sem_a).wait()
            pltpu.async_copy(y_hbm.at[src_v.at[b]], rows_b, sem_b)
            pltpu.sync_copy(rows_a, acc.at[dst_v.at[a]], add=True)
            pltpu.make_async_copy(y_hbm.at[src_v.at[b]], rows_b, sem_b).wait()
            pltpu.async_copy(y_hbm.at[src_v.at[a + 2]], rows_a, sem_a)
            pltpu.sync_copy(rows_b, acc.at[dst_v.at[b]], add=True)
            return c

        lax.fori_loop(0, KH, body2, 0)
        pltpu.make_async_copy(y_hbm.at[src_v.at[K - 1]], rows_a, sem_a).wait()
        pltpu.sync_copy(rows_a, acc.at[dst_v.at[K - 1]], add=True)
        plsc.subcore_barrier()

        pltpu.sync_copy(acc.at[pl.ds(sid * RPT, RPT)],
                        out_hbm.at[pl.ds(cid * N + sid * RPT, RPT)])

    return functools.partial(
        pl.kernel,
        body,
        out_type=jax.ShapeDtypeStruct((NC * N, d), jnp.float32),
        mesh=_MESH,
        scratch_types=[
            pltpu.VMEM((2, K, CHUNK), jnp.int32),
            pltpu.VMEM((CHUNK, d), jnp.float32),
            pltpu.VMEM((CHUNK, d), jnp.float32),
            pltpu.VMEM_SHARED((N, d), jnp.float32),
            pltpu.SemaphoreType.DMA,
            pltpu.SemaphoreType.DMA,
            pltpu.SemaphoreType.DMA,
            pltpu.SemaphoreType.DMA,
        ],
        compiler_params=_SC_PARAMS,
    )()


_sc_scatter_h = _make_sc_scatter(D_H)
_sc_scatter_o = _make_sc_scatter(D_OUT)

B_R = 2000
_GRID = (N // B_R,)


def _rsqrt_col(a_ref, b_ref):
    return lax.rsqrt(jnp.maximum(a_ref[0] + b_ref[0], 1.0))


def _tc0_body(x_ref, doa_ref, dob_ref, w_ref, o_ref):
    s = _rsqrt_col(doa_ref, dob_ref)
    o_ref[...] = jnp.dot(x_ref[...] * s, w_ref[...],
                         preferred_element_type=jnp.float32)


def _tc_mid_body(p0_ref, p1_ref, dia_ref, dib_ref, b_ref, doa_ref, dob_ref,
                 w_ref, o_ref):
    si = _rsqrt_col(dia_ref, dib_ref)
    h = (p0_ref[...] + p1_ref[...]) * si + b_ref[...]
    h = jnp.maximum(h, 0.0)
    so = _rsqrt_col(doa_ref, dob_ref)
    o_ref[...] = jnp.dot(h * so, w_ref[...],
                         preferred_element_type=jnp.float32)


def _tc_last_body(p0_ref, p1_ref, dia_ref, dib_ref, b_ref, o_ref):
    si = _rsqrt_col(dia_ref, dib_ref)
    o_ref[...] = (p0_ref[...] + p1_ref[...]) * si + b_ref[...]


def _row_spec(d):
    return pl.BlockSpec((B_R, d), lambda i: (i, 0))


def _half_spec(d, half):
    g = N // B_R
    return pl.BlockSpec((B_R, d), lambda i, _g=g, _h=half: (i + _h * _g, 0))


def _full_spec(r, c):
    return pl.BlockSpec((r, c), lambda i: (0, 0))


def _deg_spec(row):
    return pl.BlockSpec((1, B_R, 1), lambda i, _r=row: (_r, i, 0))


def _tc0(x, deg4, w):
    d_in, d_out = w.shape
    return pl.pallas_call(
        _tc0_body,
        grid=_GRID,
        in_specs=[_row_spec(d_in), _deg_spec(0), _deg_spec(2),
                  _full_spec(d_in, d_out)],
        out_specs=_row_spec(d_out),
        out_shape=jax.ShapeDtypeStruct((N, d_out), jnp.float32),
    )(x, deg4, deg4, w)


def _tc_mid(p, deg4, b, w):
    d_in, d_out = w.shape
    return pl.pallas_call(
        _tc_mid_body,
        grid=_GRID,
        in_specs=[_half_spec(d_in, 0), _half_spec(d_in, 1),
                  _deg_spec(1), _deg_spec(3), _full_spec(1, d_in),
                  _deg_spec(0), _deg_spec(2), _full_spec(d_in, d_out)],
        out_specs=_row_spec(d_out),
        out_shape=jax.ShapeDtypeStruct((N, d_out), jnp.float32),
    )(p, p, deg4, deg4, b, deg4, deg4, w)


def _tc_last(p, deg4, b):
    d = p.shape[1]
    return pl.pallas_call(
        _tc_last_body,
        grid=_GRID,
        in_specs=[_half_spec(d, 0), _half_spec(d, 1),
                  _deg_spec(1), _deg_spec(3), _full_spec(1, d)],
        out_specs=_row_spec(d),
        out_shape=jax.ShapeDtypeStruct((N, d), jnp.float32),
    )(p, p, deg4, deg4, b)


def kernel(features, edge_index, W0, b0, W1, b1, W2, b2):
    eidx = edge_index.reshape(2, NW, K, CHUNK)

    ones_c = jnp.ones((CHUNK,), jnp.float32)
    zrow = jnp.zeros((1000,), jnp.float32)
    zrows_h = jnp.zeros((NS, RPT, D_H), jnp.float32)
    zrows_o = jnp.zeros((NS, RPT, D_OUT), jnp.float32)

    deg4 = _deg_call(eidx, ones_c, zrow).reshape(4, N, 1)

    y = _tc0(features, deg4, W0)
    p = _sc_scatter_h(eidx, y, zrows_h)
    y = _tc_mid(p, deg4, b0.reshape(1, D_H), W1)
    p = _sc_scatter_h(eidx, y, zrows_h)
    y = _tc_mid(p, deg4, b1.reshape(1, D_H), W2)
    p = _sc_scatter_o(eidx, y, zrows_o)
    return _tc_last(p, deg4, b2.reshape(1, D_OUT))


# speedup vs baseline: 4.7809x; 1.3912x over previous
import functools

import jax
import jax.numpy as jnp
from jax import lax
from jax.experimental import pallas as pl
from jax.experimental.pallas import tpu as pltpu
from jax.experimental.pallas import tpu_sc as plsc

N = 10000
E = 320000
D_IN = 128
D_H = 128
D_OUT = 64

NC = 2
NS = 16
NW = NC * NS

RPT = N // NS
CHUNK = 80
K = 125
KH = (K - 1) // 2
EPW = K * CHUNK
DTILES = 10

_MESH = plsc.VectorSubcoreMesh(core_axis_name="c", subcore_axis_name="s")
_SC_PARAMS = pltpu.CompilerParams(use_tc_tiling_on_sc=False)


def _sc_degree(eidx_hbm, ones_hbm, zrow_hbm, out_hbm,
               idx_v, ones_v, acc_o, acc_i, sem):
    cid = lax.axis_index("c")
    sid = lax.axis_index("s")
    wid = cid * NS + sid

    pltpu.sync_copy(ones_hbm, ones_v)

    @pl.when(sid < DTILES)
    def _():
        pltpu.sync_copy(zrow_hbm, acc_o.at[pl.ds(sid * 1000, 1000)])
        pltpu.sync_copy(zrow_hbm, acc_i.at[pl.ds(sid * 1000, 1000)])

    pltpu.sync_copy(eidx_hbm.at[0, wid], idx_v.at[0])
    pltpu.sync_copy(eidx_hbm.at[1, wid], idx_v.at[1])
    plsc.subcore_barrier()

    def body(j, c):
        pltpu.sync_copy(ones_v, acc_o.at[idx_v.at[0, j]], add=True)
        pltpu.sync_copy(ones_v, acc_i.at[idx_v.at[1, j]], add=True)
        return c

    lax.fori_loop(0, K, body, 0)
    plsc.subcore_barrier()

    @pl.when(sid < DTILES)
    def _():
        pltpu.sync_copy(acc_o.at[pl.ds(sid * 1000, 1000)],
                        out_hbm.at[pl.ds(cid * 2 * N + sid * 1000, 1000)])
        pltpu.sync_copy(acc_i.at[pl.ds(sid * 1000, 1000)],
                        out_hbm.at[pl.ds(cid * 2 * N + N + sid * 1000, 1000)])


_deg_call = functools.partial(
    pl.kernel,
    _sc_degree,
    out_type=jax.ShapeDtypeStruct((NC * 2 * N,), jnp.float32),
    mesh=_MESH,
    scratch_types=[
        pltpu.VMEM((2, K, CHUNK), jnp.int32),
        pltpu.VMEM((CHUNK,), jnp.float32),
        pltpu.VMEM_SHARED((N,), jnp.float32),
        pltpu.VMEM_SHARED((N,), jnp.float32),
        pltpu.SemaphoreType.DMA,
    ],
    compiler_params=_SC_PARAMS,
)()


def _make_sc_scatter(d):
    def body(eidx_hbm, y_hbm, zrows_hbm, out_hbm,
             idx_v, rows_a, rows_b, rows_c, acc,
             sem_z, sem_i, sem_a, sem_b, sem_c, sem_d, sem_e, sem_f):
        cid = lax.axis_index("c")
        sid = lax.axis_index("s")
        wid = cid * NS + sid

        pltpu.async_copy(zrows_hbm.at[sid], acc.at[pl.ds(sid * RPT, RPT)],
                         sem_z)
        pltpu.async_copy(eidx_hbm.at[0, wid], idx_v.at[0], sem_i)
        pltpu.async_copy(eidx_hbm.at[1, wid], idx_v.at[1], sem_i)
        pltpu.make_async_copy(eidx_hbm.at[0, wid], idx_v.at[0], sem_i).wait()
        pltpu.make_async_copy(eidx_hbm.at[1, wid], idx_v.at[1], sem_i).wait()
        pltpu.make_async_copy(zrows_hbm.at[sid],
                              acc.at[pl.ds(sid * RPT, RPT)], sem_z).wait()
        plsc.subcore_barrier()

        src_v = idx_v.at[0]
        dst_v = idx_v.at[1]
        rows = (rows_a, rows_b, rows_c)
        gsems = (sem_a, sem_b, sem_c)
        ssems = (sem_d, sem_e, sem_f)

        def _gather(t, u):
            pltpu.async_copy(y_hbm.at[src_v.at[t]], rows[u], gsems[u])

        def _gwait(t, u):
            pltpu.make_async_copy(y_hbm.at[src_v.at[t]], rows[u],
                                  gsems[u]).wait()

        def _scat(t, u):
            pltpu.async_copy(rows[u], acc.at[dst_v.at[t]], ssems[u], add=True)

        def _swait(t, u):
            pltpu.make_async_copy(rows[u], acc.at[dst_v.at[t]],
                                  ssems[u]).wait()

        _gather(0, 0)

        def tri(m, c):
            for u in range(3):
                t = 3 * m + u
                un = (u + 1) % 3

                @pl.when(jnp.logical_and(t >= 2, t < K))
                def _():
                    _swait(t - 2, un)

                @pl.when(t + 1 < K)
                def _():
                    _gather(t + 1, un)

                @pl.when(t < K)
                def _():
                    _gwait(t, u)
                    _scat(t, u)
            return c

        lax.fori_loop(0, (K + 2) // 3, tri, 0)
        _swait(K - 2, (K - 2) % 3)
        _swait(K - 1, (K - 1) % 3)
        plsc.subcore_barrier()

        pltpu.sync_copy(acc.at[pl.ds(sid * RPT, RPT)],
                        out_hbm.at[pl.ds(cid * N + sid * RPT, RPT)])

    return functools.partial(
        pl.kernel,
        body,
        out_type=jax.ShapeDtypeStruct((NC * N, d), jnp.float32),
        mesh=_MESH,
        scratch_types=[
            pltpu.VMEM((2, K, CHUNK), jnp.int32),
            pltpu.VMEM((CHUNK, d), jnp.float32),
            pltpu.VMEM((CHUNK, d), jnp.float32),
            pltpu.VMEM((CHUNK, d), jnp.float32),
            pltpu.VMEM_SHARED((N, d), jnp.float32),
            pltpu.SemaphoreType.DMA,
            pltpu.SemaphoreType.DMA,
            pltpu.SemaphoreType.DMA,
            pltpu.SemaphoreType.DMA,
            pltpu.SemaphoreType.DMA,
            pltpu.SemaphoreType.DMA,
            pltpu.SemaphoreType.DMA,
            pltpu.SemaphoreType.DMA,
        ],
        compiler_params=_SC_PARAMS,
    )()


_sc_scatter_h = _make_sc_scatter(D_H)
_sc_scatter_o = _make_sc_scatter(D_OUT)

B_R = 2000
_GRID = (N // B_R,)


def _rsqrt_col(a_ref, b_ref):
    return lax.rsqrt(jnp.maximum(a_ref[0] + b_ref[0], 1.0))


def _tc0_body(x_ref, doa_ref, dob_ref, w_ref, o_ref):
    s = _rsqrt_col(doa_ref, dob_ref)
    o_ref[...] = jnp.dot(x_ref[...] * s, w_ref[...],
                         preferred_element_type=jnp.float32)


def _tc_mid_body(p0_ref, p1_ref, dia_ref, dib_ref, b_ref, doa_ref, dob_ref,
                 w_ref, o_ref):
    si = _rsqrt_col(dia_ref, dib_ref)
    h = (p0_ref[...] + p1_ref[...]) * si + b_ref[...]
    h = jnp.maximum(h, 0.0)
    so = _rsqrt_col(doa_ref, dob_ref)
    o_ref[...] = jnp.dot(h * so, w_ref[...],
                         preferred_element_type=jnp.float32)


def _tc_last_body(p0_ref, p1_ref, dia_ref, dib_ref, b_ref, o_ref):
    si = _rsqrt_col(dia_ref, dib_ref)
    o_ref[...] = (p0_ref[...] + p1_ref[...]) * si + b_ref[...]


def _row_spec(d):
    return pl.BlockSpec((B_R, d), lambda i: (i, 0))


def _half_spec(d, half):
    g = N // B_R
    return pl.BlockSpec((B_R, d), lambda i, _g=g, _h=half: (i + _h * _g, 0))


def _full_spec(r, c):
    return pl.BlockSpec((r, c), lambda i: (0, 0))


def _deg_spec(row):
    return pl.BlockSpec((1, B_R, 1), lambda i, _r=row: (_r, i, 0))


def _tc0(x, deg4, w):
    d_in, d_out = w.shape
    return pl.pallas_call(
        _tc0_body,
        grid=_GRID,
        in_specs=[_row_spec(d_in), _deg_spec(0), _deg_spec(2),
                  _full_spec(d_in, d_out)],
        out_specs=_row_spec(d_out),
        out_shape=jax.ShapeDtypeStruct((N, d_out), jnp.float32),
    )(x, deg4, deg4, w)


def _tc_mid(p, deg4, b, w):
    d_in, d_out = w.shape
    return pl.pallas_call(
        _tc_mid_body,
        grid=_GRID,
        in_specs=[_half_spec(d_in, 0), _half_spec(d_in, 1),
                  _deg_spec(1), _deg_spec(3), _full_spec(1, d_in),
                  _deg_spec(0), _deg_spec(2), _full_spec(d_in, d_out)],
        out_specs=_row_spec(d_out),
        out_shape=jax.ShapeDtypeStruct((N, d_out), jnp.float32),
    )(p, p, deg4, deg4, b, deg4, deg4, w)


def _tc_last(p, deg4, b):
    d = p.shape[1]
    return pl.pallas_call(
        _tc_last_body,
        grid=_GRID,
        in_specs=[_half_spec(d, 0), _half_spec(d, 1),
                  _deg_spec(1), _deg_spec(3), _full_spec(1, d)],
        out_specs=_row_spec(d),
        out_shape=jax.ShapeDtypeStruct((N, d), jnp.float32),
    )(p, p, deg4, deg4, b)


def kernel(features, edge_index, W0, b0, W1, b1, W2, b2):
    eidx = edge_index.reshape(2, NW, K, CHUNK)

    ones_c = jnp.ones((CHUNK,), jnp.float32)
    zrow = jnp.zeros((1000,), jnp.float32)
    zrows_h = jnp.zeros((NS, RPT, D_H), jnp.float32)
    zrows_o = jnp.zeros((NS, RPT, D_OUT), jnp.float32)

    deg4 = _deg_call(eidx, ones_c, zrow).reshape(4, N, 1)

    y = _tc0(features, deg4, W0)
    p = _sc_scatter_h(eidx, y, zrows_h)
    y = _tc_mid(p, deg4, b0.reshape(1, D_H), W1)
    p = _sc_scatter_h(eidx, y, zrows_h)
    y = _tc_mid(p, deg4, b1.reshape(1, D_H), W2)
    p = _sc_scatter_o(eidx, y, zrows_o)
    return _tc_last(p, deg4, b2.reshape(1, D_OUT))


# speedup vs baseline: 4.8991x; 1.0247x over previous
import functools

import jax
import jax.numpy as jnp
from jax import lax
from jax.experimental import pallas as pl
from jax.experimental.pallas import tpu as pltpu
from jax.experimental.pallas import tpu_sc as plsc

N = 10000
E = 320000
D_IN = 128
D_H = 128
D_OUT = 64

NC = 2
NS = 16
NW = NC * NS

RPT = N // NS
CHUNK = 80
K = 125
KH = (K - 1) // 2
EPW = K * CHUNK
DTILES = 10

_MESH = plsc.VectorSubcoreMesh(core_axis_name="c", subcore_axis_name="s")
_SC_PARAMS = pltpu.CompilerParams(use_tc_tiling_on_sc=False)


def _sc_degree(eidx_hbm, ones_hbm, zrow_hbm, out_hbm,
               idx_v, ones_v, acc_o, acc_i, sem):
    cid = lax.axis_index("c")
    sid = lax.axis_index("s")
    wid = cid * NS + sid

    pltpu.sync_copy(ones_hbm, ones_v)

    @pl.when(sid < DTILES)
    def _():
        pltpu.sync_copy(zrow_hbm, acc_o.at[pl.ds(sid * 1000, 1000)])
        pltpu.sync_copy(zrow_hbm, acc_i.at[pl.ds(sid * 1000, 1000)])

    pltpu.sync_copy(eidx_hbm.at[0, wid], idx_v.at[0])
    pltpu.sync_copy(eidx_hbm.at[1, wid], idx_v.at[1])
    plsc.subcore_barrier()

    def body(j, c):
        pltpu.sync_copy(ones_v, acc_o.at[idx_v.at[0, j]], add=True)
        pltpu.sync_copy(ones_v, acc_i.at[idx_v.at[1, j]], add=True)
        return c

    lax.fori_loop(0, K, body, 0)
    plsc.subcore_barrier()

    @pl.when(sid < DTILES)
    def _():
        pltpu.sync_copy(acc_o.at[pl.ds(sid * 1000, 1000)],
                        out_hbm.at[pl.ds(cid * 2 * N + sid * 1000, 1000)])
        pltpu.sync_copy(acc_i.at[pl.ds(sid * 1000, 1000)],
                        out_hbm.at[pl.ds(cid * 2 * N + N + sid * 1000, 1000)])


_deg_call = functools.partial(
    pl.kernel,
    _sc_degree,
    out_type=jax.ShapeDtypeStruct((NC * 2 * N,), jnp.float32),
    mesh=_MESH,
    scratch_types=[
        pltpu.VMEM((2, K, CHUNK), jnp.int32),
        pltpu.VMEM((CHUNK,), jnp.float32),
        pltpu.VMEM_SHARED((N,), jnp.float32),
        pltpu.VMEM_SHARED((N,), jnp.float32),
        pltpu.SemaphoreType.DMA,
    ],
    compiler_params=_SC_PARAMS,
)()


def _make_sc_scatter(d):
    def body(eidx_hbm, y_hbm, zrows_hbm, out_hbm,
             idx_v, rows_a, rows_b, rows_c, acc,
             sem_z, sem_i, sem_a, sem_b, sem_c, sem_d, sem_e, sem_f):
        cid = lax.axis_index("c")
        sid = lax.axis_index("s")
        wid = cid * NS + sid

        pltpu.async_copy(zrows_hbm.at[sid], acc.at[pl.ds(sid * RPT, RPT)],
                         sem_z)
        pltpu.async_copy(eidx_hbm.at[0, wid], idx_v.at[0], sem_i)
        pltpu.async_copy(eidx_hbm.at[1, wid], idx_v.at[1], sem_i)
        pltpu.make_async_copy(eidx_hbm.at[0, wid], idx_v.at[0], sem_i).wait()
        pltpu.make_async_copy(eidx_hbm.at[1, wid], idx_v.at[1], sem_i).wait()
        pltpu.make_async_copy(zrows_hbm.at[sid],
                              acc.at[pl.ds(sid * RPT, RPT)], sem_z).wait()
        plsc.subcore_barrier()

        src_v = idx_v.at[0]
        dst_v = idx_v.at[1]
        rows = (rows_a, rows_b, rows_c)
        gsems = (sem_a, sem_b, sem_c)
        ssems = (sem_d, sem_e, sem_f)

        def _gather(t, u):
            pltpu.async_copy(y_hbm.at[src_v.at[t]], rows[u], gsems[u])

        def _gwait(t, u):
            pltpu.make_async_copy(y_hbm.at[src_v.at[t]], rows[u],
                                  gsems[u]).wait()

        def _scat(t, u):
            pltpu.async_copy(rows[u], acc.at[dst_v.at[t]], ssems[u], add=True)

        def _swait(t, u):
            pltpu.make_async_copy(rows[u], acc.at[dst_v.at[t]],
                                  ssems[u]).wait()

        _gather(0, 0)

        def tri(m, c):
            for u in range(3):
                t = 3 * m + u
                un = (u + 1) % 3

                @pl.when(jnp.logical_and(t >= 2, t < K))
                def _():
                    _swait(t - 2, un)

                @pl.when(t + 1 < K)
                def _():
                    _gather(t + 1, un)

                @pl.when(t < K)
                def _():
                    _gwait(t, u)
                    _scat(t, u)
            return c

        lax.fori_loop(0, (K + 2) // 3, tri, 0)
        _swait(K - 2, (K - 2) % 3)
        _swait(K - 1, (K - 1) % 3)
        plsc.subcore_barrier()

        pltpu.sync_copy(acc.at[pl.ds(sid * RPT, RPT)],
                        out_hbm.at[pl.ds(cid * N + sid * RPT, RPT)])

    return functools.partial(
        pl.kernel,
        body,
        out_type=jax.ShapeDtypeStruct((NC * N, d), jnp.float32),
        mesh=_MESH,
        scratch_types=[
            pltpu.VMEM((2, K, CHUNK), jnp.int32),
            pltpu.VMEM((CHUNK, d), jnp.float32),
            pltpu.VMEM((CHUNK, d), jnp.float32),
            pltpu.VMEM((CHUNK, d), jnp.float32),
            pltpu.VMEM_SHARED((N, d), jnp.float32),
            pltpu.SemaphoreType.DMA,
            pltpu.SemaphoreType.DMA,
            pltpu.SemaphoreType.DMA,
            pltpu.SemaphoreType.DMA,
            pltpu.SemaphoreType.DMA,
            pltpu.SemaphoreType.DMA,
            pltpu.SemaphoreType.DMA,
            pltpu.SemaphoreType.DMA,
        ],
        compiler_params=_SC_PARAMS,
    )()


_sc_scatter_h = _make_sc_scatter(D_H)
_sc_scatter_o = _make_sc_scatter(D_OUT)

B_R = 2000
_GRID = (N // B_R,)


def _rsqrt_col(a_ref, b_ref):
    return lax.rsqrt(jnp.maximum(a_ref[...] + b_ref[...], 1.0))


def _tc0_body(x_ref, doa_ref, dob_ref, w_ref, o_ref):
    s = _rsqrt_col(doa_ref, dob_ref)
    o_ref[...] = jnp.dot(x_ref[...] * s, w_ref[...],
                         preferred_element_type=jnp.float32)


def _tc_mid_body(p0_ref, p1_ref, dia_ref, dib_ref, b_ref, doa_ref, dob_ref,
                 w_ref, o_ref):
    si = _rsqrt_col(dia_ref, dib_ref)
    h = (p0_ref[...] + p1_ref[...]) * si + b_ref[...]
    h = jnp.maximum(h, 0.0)
    so = _rsqrt_col(doa_ref, dob_ref)
    o_ref[...] = jnp.dot(h * so, w_ref[...],
                         preferred_element_type=jnp.float32)


def _tc_last_body(p0_ref, p1_ref, dia_ref, dib_ref, b_ref, o_ref):
    si = _rsqrt_col(dia_ref, dib_ref)
    o_ref[...] = (p0_ref[...] + p1_ref[...]) * si + b_ref[...]


def _row_spec(d):
    return pl.BlockSpec((B_R, d), lambda i: (i, 0))


def _half_spec(d, half):
    g = N // B_R
    return pl.BlockSpec((B_R, d), lambda i, _g=g, _h=half: (i + _h * _g, 0))


def _full_spec(r, c):
    return pl.BlockSpec((r, c), lambda i: (0, 0))


_VEC = pl.BlockSpec((B_R, 1), lambda i: (i, 0))


def _tc0(x, doa, dob, w):
    d_in, d_out = w.shape
    return pl.pallas_call(
        _tc0_body,
        grid=_GRID,
        in_specs=[_row_spec(d_in), _VEC, _VEC, _full_spec(d_in, d_out)],
        out_specs=_row_spec(d_out),
        out_shape=jax.ShapeDtypeStruct((N, d_out), jnp.float32),
    )(x, doa, dob, w)


def _tc_mid(p, dia, dib, b, doa, dob, w):
    d_in, d_out = w.shape
    return pl.pallas_call(
        _tc_mid_body,
        grid=_GRID,
        in_specs=[_half_spec(d_in, 0), _half_spec(d_in, 1),
                  _VEC, _VEC, _full_spec(1, d_in),
                  _VEC, _VEC, _full_spec(d_in, d_out)],
        out_specs=_row_spec(d_out),
        out_shape=jax.ShapeDtypeStruct((N, d_out), jnp.float32),
    )(p, p, dia, dib, b, doa, dob, w)


def _tc_last(p, dia, dib, b):
    d = p.shape[1]
    return pl.pallas_call(
        _tc_last_body,
        grid=_GRID,
        in_specs=[_half_spec(d, 0), _half_spec(d, 1),
                  _VEC, _VEC, _full_spec(1, d)],
        out_specs=_row_spec(d),
        out_shape=jax.ShapeDtypeStruct((N, d), jnp.float32),
    )(p, p, dia, dib, b)


def kernel(features, edge_index, W0, b0, W1, b1, W2, b2):
    eidx = edge_index.reshape(2, NW, K, CHUNK)

    ones_c = jnp.ones((CHUNK,), jnp.float32)
    zrow = jnp.zeros((1000,), jnp.float32)
    zrows_h = jnp.zeros((NS, RPT, D_H), jnp.float32)
    zrows_o = jnp.zeros((NS, RPT, D_OUT), jnp.float32)

    deg = _deg_call(eidx, ones_c, zrow)
    doa = deg[0 * N:1 * N].reshape(N, 1)
    dia = deg[1 * N:2 * N].reshape(N, 1)
    dob = deg[2 * N:3 * N].reshape(N, 1)
    dib = deg[3 * N:4 * N].reshape(N, 1)

    y = _tc0(features, doa, dob, W0)
    p = _sc_scatter_h(eidx, y, zrows_h)
    y = _tc_mid(p, dia, dib, b0.reshape(1, D_H), doa, dob, W1)
    p = _sc_scatter_h(eidx, y, zrows_h)
    y = _tc_mid(p, dia, dib, b1.reshape(1, D_H), doa, dob, W2)
    p = _sc_scatter_o(eidx, y, zrows_o)
    return _tc_last(p, dia, dib, b2.reshape(1, D_OUT))


# speedup vs baseline: 5.2378x; 1.0691x over previous
import functools

import jax
import jax.numpy as jnp
from jax import lax
from jax.experimental import pallas as pl
from jax.experimental.pallas import tpu as pltpu
from jax.experimental.pallas import tpu_sc as plsc

N = 10000
E = 320000
D_IN = 128
D_H = 128
D_OUT = 64

NC = 2
NS = 16
NW = NC * NS

RPT = N // NS
CHUNK = 80
K = 125
KH = (K - 1) // 2
EPW = K * CHUNK
DTILES = 10

_MESH = plsc.VectorSubcoreMesh(core_axis_name="c", subcore_axis_name="s")
_SC_PARAMS = pltpu.CompilerParams(use_tc_tiling_on_sc=False)


def _sc_degree(eidx_hbm, ones_hbm, zrow_hbm, out_hbm,
               idx_v, ones_v, acc_o, acc_i, so0, si0, so1, si1):
    cid = lax.axis_index("c")
    sid = lax.axis_index("s")
    wid = cid * NS + sid

    pltpu.sync_copy(ones_hbm, ones_v)

    @pl.when(sid < DTILES)
    def _():
        pltpu.sync_copy(zrow_hbm, acc_o.at[pl.ds(sid * 1000, 1000)])
        pltpu.sync_copy(zrow_hbm, acc_i.at[pl.ds(sid * 1000, 1000)])

    pltpu.sync_copy(eidx_hbm.at[0, wid], idx_v.at[0])
    pltpu.sync_copy(eidx_hbm.at[1, wid], idx_v.at[1])
    plsc.subcore_barrier()

    def _dscat(j, p, sem):
        pltpu.async_copy(ones_v, (acc_o if p == 0 else acc_i).at[idx_v.at[p, j]],
                         sem, add=True)

    def _dwait(j, p, sem):
        pltpu.make_async_copy(ones_v,
                              (acc_o if p == 0 else acc_i).at[idx_v.at[p, j]],
                              sem).wait()

    sems = (so0, si0, so1, si1)
    for u in range(2):
        _dscat(u, 0, sems[2 * u])
        _dscat(u, 1, sems[2 * u + 1])

    def pair(m, c):
        for u in range(2):
            j = 2 * m + u

            @pl.when(j + 2 < K)
            def _():
                _dwait(j, 0, sems[2 * u])
                _dscat(j + 2, 0, sems[2 * u])
                _dwait(j, 1, sems[2 * u + 1])
                _dscat(j + 2, 1, sems[2 * u + 1])
        return c

    lax.fori_loop(0, (K + 1) // 2, pair, 0)
    _dwait(K - 2, 0, sems[2 * ((K - 2) % 2)])
    _dwait(K - 2, 1, sems[2 * ((K - 2) % 2) + 1])
    _dwait(K - 1, 0, sems[2 * ((K - 1) % 2)])
    _dwait(K - 1, 1, sems[2 * ((K - 1) % 2) + 1])
    plsc.subcore_barrier()

    @pl.when(sid < DTILES)
    def _():
        pltpu.sync_copy(acc_o.at[pl.ds(sid * 1000, 1000)],
                        out_hbm.at[pl.ds(cid * 2 * N + sid * 1000, 1000)])
        pltpu.sync_copy(acc_i.at[pl.ds(sid * 1000, 1000)],
                        out_hbm.at[pl.ds(cid * 2 * N + N + sid * 1000, 1000)])


_deg_call = functools.partial(
    pl.kernel,
    _sc_degree,
    out_type=jax.ShapeDtypeStruct((NC * 2 * N,), jnp.float32),
    mesh=_MESH,
    scratch_types=[
        pltpu.VMEM((2, K, CHUNK), jnp.int32),
        pltpu.VMEM((CHUNK,), jnp.float32),
        pltpu.VMEM_SHARED((N,), jnp.float32),
        pltpu.VMEM_SHARED((N,), jnp.float32),
        pltpu.SemaphoreType.DMA,
        pltpu.SemaphoreType.DMA,
        pltpu.SemaphoreType.DMA,
        pltpu.SemaphoreType.DMA,
    ],
    compiler_params=_SC_PARAMS,
)()


def _make_sc_scatter(d):
    def body(eidx_hbm, y_hbm, zrows_hbm, out_hbm,
             idx_v, rows_a, rows_b, rows_c, acc,
             sem_z, sem_i, sem_a, sem_b, sem_c, sem_d, sem_e, sem_f):
        cid = lax.axis_index("c")
        sid = lax.axis_index("s")
        wid = cid * NS + sid

        pltpu.async_copy(zrows_hbm.at[sid], acc.at[pl.ds(sid * RPT, RPT)],
                         sem_z)
        pltpu.async_copy(eidx_hbm.at[0, wid], idx_v.at[0], sem_i)
        pltpu.async_copy(eidx_hbm.at[1, wid], idx_v.at[1], sem_i)
        pltpu.make_async_copy(eidx_hbm.at[0, wid], idx_v.at[0], sem_i).wait()
        pltpu.make_async_copy(eidx_hbm.at[1, wid], idx_v.at[1], sem_i).wait()
        pltpu.make_async_copy(zrows_hbm.at[sid],
                              acc.at[pl.ds(sid * RPT, RPT)], sem_z).wait()
        plsc.subcore_barrier()

        src_v = idx_v.at[0]
        dst_v = idx_v.at[1]
        rows = (rows_a, rows_b, rows_c)
        gsems = (sem_a, sem_b, sem_c)
        ssems = (sem_d, sem_e, sem_f)

        def _gather(t, u):
            pltpu.async_copy(y_hbm.at[src_v.at[t]], rows[u], gsems[u])

        def _gwait(t, u):
            pltpu.make_async_copy(y_hbm.at[src_v.at[t]], rows[u],
                                  gsems[u]).wait()

        def _scat(t, u):
            pltpu.async_copy(rows[u], acc.at[dst_v.at[t]], ssems[u], add=True)

        def _swait(t, u):
            pltpu.make_async_copy(rows[u], acc.at[dst_v.at[t]],
                                  ssems[u]).wait()

        _gather(0, 0)

        def tri(m, c):
            for u in range(3):
                t = 3 * m + u
                un = (u + 1) % 3

                @pl.when(jnp.logical_and(t >= 2, t < K))
                def _():
                    _swait(t - 2, un)

                @pl.when(t + 1 < K)
                def _():
                    _gather(t + 1, un)

                @pl.when(t < K)
                def _():
                    _gwait(t, u)
                    _scat(t, u)
            return c

        lax.fori_loop(0, (K + 2) // 3, tri, 0)
        _swait(K - 2, (K - 2) % 3)
        _swait(K - 1, (K - 1) % 3)
        plsc.subcore_barrier()

        pltpu.sync_copy(acc.at[pl.ds(sid * RPT, RPT)],
                        out_hbm.at[pl.ds(cid * N + sid * RPT, RPT)])

    return functools.partial(
        pl.kernel,
        body,
        out_type=jax.ShapeDtypeStruct((NC * N, d), jnp.float32),
        mesh=_MESH,
        scratch_types=[
            pltpu.VMEM((2, K, CHUNK), jnp.int32),
            pltpu.VMEM((CHUNK, d), jnp.float32),
            pltpu.VMEM((CHUNK, d), jnp.float32),
            pltpu.VMEM((CHUNK, d), jnp.float32),
            pltpu.VMEM_SHARED((N, d), jnp.float32),
            pltpu.SemaphoreType.DMA,
            pltpu.SemaphoreType.DMA,
            pltpu.SemaphoreType.DMA,
            pltpu.SemaphoreType.DMA,
            pltpu.SemaphoreType.DMA,
            pltpu.SemaphoreType.DMA,
            pltpu.SemaphoreType.DMA,
            pltpu.SemaphoreType.DMA,
        ],
        compiler_params=_SC_PARAMS,
    )()


_sc_scatter_h = _make_sc_scatter(D_H)
_sc_scatter_o = _make_sc_scatter(D_OUT)

B_R = 2000
_GRID = (N // B_R,)


def _rsqrt_col(a_ref, b_ref):
    return lax.rsqrt(jnp.maximum(a_ref[...] + b_ref[...], 1.0))


def _tc0_body(x_ref, doa_ref, dob_ref, w_ref, o_ref):
    s = _rsqrt_col(doa_ref, dob_ref)
    o_ref[...] = jnp.dot(x_ref[...] * s, w_ref[...],
                         preferred_element_type=jnp.float32)


def _tc_mid_body(p0_ref, p1_ref, dia_ref, dib_ref, b_ref, doa_ref, dob_ref,
                 w_ref, o_ref):
    si = _rsqrt_col(dia_ref, dib_ref)
    h = (p0_ref[...] + p1_ref[...]) * si + b_ref[...]
    h = jnp.maximum(h, 0.0)
    so = _rsqrt_col(doa_ref, dob_ref)
    o_ref[...] = jnp.dot(h * so, w_ref[...],
                         preferred_element_type=jnp.float32)


def _tc_last_body(q0_ref, q1_ref, dia_ref, dib_ref, b_ref, o_ref):
    s2 = lax.rsqrt(jnp.maximum(dia_ref[...] + dib_ref[...], 1.0))
    lane = lax.broadcasted_iota(jnp.int32, (B2, 2 * D_OUT), 1)
    s = jnp.where(lane < D_OUT, s2[:, 0:1], s2[:, 1:2])
    o_ref[...] = (q0_ref[...] + q1_ref[...]) * s + b_ref[...]


def _row_spec(d):
    return pl.BlockSpec((B_R, d), lambda i: (i, 0))


def _half_spec(d, half):
    g = N // B_R
    return pl.BlockSpec((B_R, d), lambda i, _g=g, _h=half: (i + _h * _g, 0))


def _full_spec(r, c):
    return pl.BlockSpec((r, c), lambda i: (0, 0))


_VEC = pl.BlockSpec((B_R, 1), lambda i: (i, 0))


def _tc0(x, doa, dob, w):
    d_in, d_out = w.shape
    return pl.pallas_call(
        _tc0_body,
        grid=_GRID,
        in_specs=[_row_spec(d_in), _VEC, _VEC, _full_spec(d_in, d_out)],
        out_specs=_row_spec(d_out),
        out_shape=jax.ShapeDtypeStruct((N, d_out), jnp.float32),
    )(x, doa, dob, w)


def _tc_mid(p, dia, dib, b, doa, dob, w):
    d_in, d_out = w.shape
    return pl.pallas_call(
        _tc_mid_body,
        grid=_GRID,
        in_specs=[_half_spec(d_in, 0), _half_spec(d_in, 1),
                  _VEC, _VEC, _full_spec(1, d_in),
                  _VEC, _VEC, _full_spec(d_in, d_out)],
        out_specs=_row_spec(d_out),
        out_shape=jax.ShapeDtypeStruct((N, d_out), jnp.float32),
    )(p, p, dia, dib, b, doa, dob, w)


B2 = B_R // 2


def _tc_last(q, dia2, dib2, b128):
    g = N // B_R
    spec_q = pl.BlockSpec((B2, 2 * D_OUT), lambda i: (i, 0))
    spec_q1 = pl.BlockSpec((B2, 2 * D_OUT), lambda i, _g=g: (i + _g, 0))
    spec_d = pl.BlockSpec((B2, 2), lambda i: (i, 0))
    return pl.pallas_call(
        _tc_last_body,
        grid=_GRID,
        in_specs=[spec_q, spec_q1, spec_d, spec_d,
                  _full_spec(1, 2 * D_OUT)],
        out_specs=spec_q,
        out_shape=jax.ShapeDtypeStruct((N // 2, 2 * D_OUT), jnp.float32),
    )(q, q, dia2, dib2, b128)


def kernel(features, edge_index, W0, b0, W1, b1, W2, b2):
    eidx = edge_index.reshape(2, NW, K, CHUNK)

    ones_c = jnp.ones((CHUNK,), jnp.float32)
    zrow = jnp.zeros((1000,), jnp.float32)
    zrows_h = jnp.zeros((NS, RPT, D_H), jnp.float32)
    zrows_o = jnp.zeros((NS, RPT, D_OUT), jnp.float32)

    deg = _deg_call(eidx, ones_c, zrow)
    doa = deg[0 * N:1 * N].reshape(N, 1)
    dia = deg[1 * N:2 * N].reshape(N, 1)
    dob = deg[2 * N:3 * N].reshape(N, 1)
    dib = deg[3 * N:4 * N].reshape(N, 1)
    dia2 = deg[1 * N:2 * N].reshape(N // 2, 2)
    dib2 = deg[3 * N:4 * N].reshape(N // 2, 2)

    y = _tc0(features, doa, dob, W0)
    p = _sc_scatter_h(eidx, y, zrows_h)
    y = _tc_mid(p, dia, dib, b0.reshape(1, D_H), doa, dob, W1)
    p = _sc_scatter_h(eidx, y, zrows_h)
    y = _tc_mid(p, dia, dib, b1.reshape(1, D_H), doa, dob, W2)
    p = _sc_scatter_o(eidx, y, zrows_o)
    q = p.reshape(N, 2 * D_OUT)
    b128 = jnp.concatenate([b2, b2]).reshape(1, 2 * D_OUT)
    return _tc_last(q, dia2, dib2, b128).reshape(N, D_OUT)


# speedup vs baseline: 5.5011x; 1.0503x over previous
import functools

import jax
import jax.numpy as jnp
from jax import lax
from jax.experimental import pallas as pl
from jax.experimental.pallas import tpu as pltpu
from jax.experimental.pallas import tpu_sc as plsc

N = 10000
E = 320000
D_IN = 128
D_H = 128
D_OUT = 64

NC = 2
NS = 16
NW = NC * NS

RPT = N // NS
CHUNK = 80
K = 125
KH = (K - 1) // 2
EPW = K * CHUNK
DTILES = 10
N_ROW = 10240
SEG = 10240

_MESH = plsc.VectorSubcoreMesh(core_axis_name="c", subcore_axis_name="s")
_SC_PARAMS = pltpu.CompilerParams(use_tc_tiling_on_sc=False)


def _sc_degree(eidx_hbm, ones_hbm, zrow_hbm, out_hbm,
               idx_v, ones_v, acc_o, acc_i, so0, si0, so1, si1):
    cid = lax.axis_index("c")
    sid = lax.axis_index("s")
    wid = cid * NS + sid

    pltpu.sync_copy(ones_hbm, ones_v)

    @pl.when(sid < DTILES)
    def _():
        pltpu.sync_copy(zrow_hbm, acc_o.at[pl.ds(sid * 1000, 1000)])
        pltpu.sync_copy(zrow_hbm, acc_i.at[pl.ds(sid * 1000, 1000)])

    pltpu.sync_copy(eidx_hbm.at[0, wid], idx_v.at[0])
    pltpu.sync_copy(eidx_hbm.at[1, wid], idx_v.at[1])
    plsc.subcore_barrier()

    def _dscat(j, p, sem):
        pltpu.async_copy(ones_v, (acc_o if p == 0 else acc_i).at[idx_v.at[p, j]],
                         sem, add=True)

    def _dwait(j, p, sem):
        pltpu.make_async_copy(ones_v,
                              (acc_o if p == 0 else acc_i).at[idx_v.at[p, j]],
                              sem).wait()

    sems = (so0, si0, so1, si1)
    for u in range(2):
        _dscat(u, 0, sems[2 * u])
        _dscat(u, 1, sems[2 * u + 1])

    def pair(m, c):
        for u in range(2):
            j = 2 * m + u

            @pl.when(j + 2 < K)
            def _():
                _dwait(j, 0, sems[2 * u])
                _dscat(j + 2, 0, sems[2 * u])
                _dwait(j, 1, sems[2 * u + 1])
                _dscat(j + 2, 1, sems[2 * u + 1])
        return c

    lax.fori_loop(0, (K + 1) // 2, pair, 0)
    _dwait(K - 2, 0, sems[2 * ((K - 2) % 2)])
    _dwait(K - 2, 1, sems[2 * ((K - 2) % 2) + 1])
    _dwait(K - 1, 0, sems[2 * ((K - 1) % 2)])
    _dwait(K - 1, 1, sems[2 * ((K - 1) % 2) + 1])
    plsc.subcore_barrier()

    @pl.when(sid < DTILES)
    def _():
        pltpu.sync_copy(acc_o.at[pl.ds(sid * 1000, 1000)],
                        out_hbm.at[pl.ds(cid * 2 * SEG + sid * 1000, 1000)])
        pltpu.sync_copy(acc_i.at[pl.ds(sid * 1000, 1000)],
                        out_hbm.at[pl.ds(cid * 2 * SEG + SEG + sid * 1000,
                                         1000)])


_deg_call = functools.partial(
    pl.kernel,
    _sc_degree,
    out_type=jax.ShapeDtypeStruct((4 * SEG,), jnp.float32),
    mesh=_MESH,
    scratch_types=[
        pltpu.VMEM((2, K, CHUNK), jnp.int32),
        pltpu.VMEM((CHUNK,), jnp.float32),
        pltpu.VMEM_SHARED((N,), jnp.float32),
        pltpu.VMEM_SHARED((N,), jnp.float32),
        pltpu.SemaphoreType.DMA,
        pltpu.SemaphoreType.DMA,
        pltpu.SemaphoreType.DMA,
        pltpu.SemaphoreType.DMA,
    ],
    compiler_params=_SC_PARAMS,
)()


def _make_sc_scatter(d):
    def body(eidx_hbm, y_hbm, zrows_hbm, out_hbm,
             idx_v, rows_a, rows_b, rows_c, acc,
             sem_z, sem_i, sem_a, sem_b, sem_c, sem_d, sem_e, sem_f):
        cid = lax.axis_index("c")
        sid = lax.axis_index("s")
        wid = cid * NS + sid

        pltpu.async_copy(zrows_hbm.at[sid], acc.at[pl.ds(sid * RPT, RPT)],
                         sem_z)
        pltpu.async_copy(eidx_hbm.at[0, wid], idx_v.at[0], sem_i)
        pltpu.async_copy(eidx_hbm.at[1, wid], idx_v.at[1], sem_i)
        pltpu.make_async_copy(eidx_hbm.at[0, wid], idx_v.at[0], sem_i).wait()
        pltpu.make_async_copy(eidx_hbm.at[1, wid], idx_v.at[1], sem_i).wait()
        pltpu.make_async_copy(zrows_hbm.at[sid],
                              acc.at[pl.ds(sid * RPT, RPT)], sem_z).wait()
        plsc.subcore_barrier()

        src_v = idx_v.at[0]
        dst_v = idx_v.at[1]
        rows = (rows_a, rows_b, rows_c)
        gsems = (sem_a, sem_b, sem_c)
        ssems = (sem_d, sem_e, sem_f)

        def _gather(t, u):
            pltpu.async_copy(y_hbm.at[src_v.at[t]], rows[u], gsems[u])

        def _gwait(t, u):
            pltpu.make_async_copy(y_hbm.at[src_v.at[t]], rows[u],
                                  gsems[u]).wait()

        def _scat(t, u):
            pltpu.async_copy(rows[u], acc.at[dst_v.at[t]], ssems[u], add=True)

        def _swait(t, u):
            pltpu.make_async_copy(rows[u], acc.at[dst_v.at[t]],
                                  ssems[u]).wait()

        _gather(0, 0)

        def tri(m, c):
            for u in range(3):
                t = 3 * m + u
                un = (u + 1) % 3

                @pl.when(jnp.logical_and(t >= 2, t < K))
                def _():
                    _swait(t - 2, un)

                @pl.when(t + 1 < K)
                def _():
                    _gather(t + 1, un)

                @pl.when(t < K)
                def _():
                    _gwait(t, u)
                    _scat(t, u)
            return c

        lax.fori_loop(0, (K + 2) // 3, tri, 0)
        _swait(K - 2, (K - 2) % 3)
        _swait(K - 1, (K - 1) % 3)
        plsc.subcore_barrier()

        pltpu.sync_copy(acc.at[pl.ds(sid * RPT, RPT)],
                        out_hbm.at[pl.ds(cid * N_ROW + sid * RPT, RPT)])

    return functools.partial(
        pl.kernel,
        body,
        out_type=jax.ShapeDtypeStruct((NC * N_ROW, d), jnp.float32),
        mesh=_MESH,
        scratch_types=[
            pltpu.VMEM((2, K, CHUNK), jnp.int32),
            pltpu.VMEM((CHUNK, d), jnp.float32),
            pltpu.VMEM((CHUNK, d), jnp.float32),
            pltpu.VMEM((CHUNK, d), jnp.float32),
            pltpu.VMEM_SHARED((N, d), jnp.float32),
            pltpu.SemaphoreType.DMA,
            pltpu.SemaphoreType.DMA,
            pltpu.SemaphoreType.DMA,
            pltpu.SemaphoreType.DMA,
            pltpu.SemaphoreType.DMA,
            pltpu.SemaphoreType.DMA,
            pltpu.SemaphoreType.DMA,
            pltpu.SemaphoreType.DMA,
        ],
        compiler_params=_SC_PARAMS,
    )()


_sc_scatter_h = _make_sc_scatter(D_H)
_sc_scatter_o = _make_sc_scatter(D_OUT)

B_R = 2560
_GRID = (N_ROW // B_R,)


def _rsqrt_col(d_ref, off_a, off_b):
    i = pl.program_id(0)
    a = d_ref[pl.ds(off_a + i * B_R, B_R)]
    b = d_ref[pl.ds(off_b + i * B_R, B_R)]
    s = lax.rsqrt(jnp.maximum(a + b, 1.0))
    return s.reshape(B_R, 1)


def _tc0_body(x_ref, d_ref, w_ref, o_ref):
    s = _rsqrt_col(d_ref, 0, 2 * SEG)
    o_ref[...] = jnp.dot(x_ref[...] * s, w_ref[...],
                         preferred_element_type=jnp.float32)


def _tc_mid_body(p0_ref, p1_ref, d_ref, b_ref, w_ref, o_ref):
    si = _rsqrt_col(d_ref, SEG, 3 * SEG)
    h = (p0_ref[...] + p1_ref[...]) * si + b_ref[...]
    h = jnp.maximum(h, 0.0)
    so = _rsqrt_col(d_ref, 0, 2 * SEG)
    o_ref[...] = jnp.dot(h * so, w_ref[...],
                         preferred_element_type=jnp.float32)


def _tc_last_body(q0_ref, q1_ref, dia_ref, dib_ref, b_ref, o_ref):
    s2 = lax.rsqrt(jnp.maximum(dia_ref[...] + dib_ref[...], 1.0))
    lane = lax.broadcasted_iota(jnp.int32, (B2, 2 * D_OUT), 1)
    s = jnp.where(lane < D_OUT, s2[:, 0:1], s2[:, 1:2])
    o_ref[...] = (q0_ref[...] + q1_ref[...]) * s + b_ref[...]


def _row_spec(d):
    return pl.BlockSpec((B_R, d), lambda i: (i, 0))


def _half_spec(d, half):
    g = N_ROW // B_R
    return pl.BlockSpec((B_R, d), lambda i, _g=g, _h=half: (i + _h * _g, 0))


def _full_spec(r, c):
    return pl.BlockSpec((r, c), lambda i: (0, 0))


_DEG_SPEC = pl.BlockSpec((4 * SEG,), lambda i: (0,))


def _tc0(x, deg, w):
    d_in, d_out = w.shape
    return pl.pallas_call(
        _tc0_body,
        grid=_GRID,
        in_specs=[_row_spec(d_in), _DEG_SPEC, _full_spec(d_in, d_out)],
        out_specs=_row_spec(d_out),
        out_shape=jax.ShapeDtypeStruct((N_ROW, d_out), jnp.float32),
    )(x, deg, w)


def _tc_mid(p, deg, b, w):
    d_in, d_out = w.shape
    return pl.pallas_call(
        _tc_mid_body,
        grid=_GRID,
        in_specs=[_half_spec(d_in, 0), _half_spec(d_in, 1),
                  _DEG_SPEC, _full_spec(1, d_in), _full_spec(d_in, d_out)],
        out_specs=_row_spec(d_out),
        out_shape=jax.ShapeDtypeStruct((N_ROW, d_out), jnp.float32),
    )(p, p, deg, b, w)


B2 = B_R // 2


def _tc_last(q, dia2, dib2, b128):
    g = N_ROW // B_R
    spec_q = pl.BlockSpec((B2, 2 * D_OUT), lambda i: (i, 0))
    spec_q1 = pl.BlockSpec((B2, 2 * D_OUT), lambda i, _g=g: (i + _g, 0))
    spec_d = pl.BlockSpec((B2, 2), lambda i: (i, 0))
    return pl.pallas_call(
        _tc_last_body,
        grid=_GRID,
        in_specs=[spec_q, spec_q1, spec_d, spec_d,
                  _full_spec(1, 2 * D_OUT)],
        out_specs=spec_q,
        out_shape=jax.ShapeDtypeStruct((N_ROW // 2, 2 * D_OUT), jnp.float32),
    )(q, q, dia2, dib2, b128)


def kernel(features, edge_index, W0, b0, W1, b1, W2, b2):
    eidx = edge_index.reshape(2, NW, K, CHUNK)

    ones_c = jnp.ones((CHUNK,), jnp.float32)
    zrow = jnp.zeros((1000,), jnp.float32)
    zrows_h = jnp.zeros((NS, RPT, D_H), jnp.float32)
    zrows_o = jnp.zeros((NS, RPT, D_OUT), jnp.float32)

    deg = _deg_call(eidx, ones_c, zrow)

    x = jnp.pad(features, ((0, N_ROW - N), (0, 0)))
    y = _tc0(x, deg, W0)
    p = _sc_scatter_h(eidx, y, zrows_h)
    y = _tc_mid(p, deg, b0.reshape(1, D_H), W1)
    p = _sc_scatter_h(eidx, y, zrows_h)
    y = _tc_mid(p, deg, b1.reshape(1, D_H), W2)
    p = _sc_scatter_o(eidx, y, zrows_o)
    q = p.reshape(N_ROW, 2 * D_OUT)
    b128 = jnp.concatenate([b2, b2]).reshape(1, 2 * D_OUT)
    dia2 = deg[SEG:2 * SEG].reshape(SEG // 2, 2)
    dib2 = deg[3 * SEG:4 * SEG].reshape(SEG // 2, 2)
    return _tc_last(q, dia2, dib2, b128).reshape(N_ROW, D_OUT)[:N]
